# baseline (speedup 1.0000x reference)
import functools

import jax
import jax.numpy as jnp
from jax import lax
from jax.experimental import pallas as pl
from jax.experimental.pallas import tpu as pltpu
from jax.experimental.pallas import tpu_sc as plsc

_BATCH = 4
_SEQ = 8192
_D = 1024
_LANES = 16
_NC = 2
_NS = 16
_NW = _NC * _NS
_ROWS_PER_W = _SEQ // _NW
_C = 4
_CHUNKS = _ROWS_PER_W // _C
_NBUF = 4


def _sc_body(x_hbm, pos_hbm, out_hbm, xbuf, pbuf, *sems):
    wid = lax.axis_index("s") * _NC + lax.axis_index("c")
    s_base = wid * _ROWS_PER_W
    insem = sems[:_NBUF]
    outsem = sems[_NBUF:]

    def start_in(c, p):
        s0 = s_base + c * _C
        pltpu.async_copy(pos_hbm.at[pl.ds(s0, _C)], pbuf.at[p], insem[p])
        pltpu.async_copy(x_hbm.at[:, pl.ds(s0, _C)], xbuf.at[p], insem[p])

    def wait_in(p):
        pltpu.make_async_copy(pos_hbm.at[pl.ds(0, _C)], pbuf.at[p],
                              insem[p]).wait()
        pltpu.make_async_copy(x_hbm.at[:, pl.ds(0, _C)], xbuf.at[p],
                              insem[p]).wait()

    def start_out(c, p):
        s0 = s_base + c * _C
        pltpu.async_copy(xbuf.at[p], out_hbm.at[:, pl.ds(s0, _C)], outsem[p])

    def wait_out(p):
        pltpu.make_async_copy(x_hbm.at[:, pl.ds(0, _C)], xbuf.at[p],
                              outsem[p]).wait()

    def compute(p):
        def batch_body(bb, c3):
            def row_body(r, c2):
                for j in range(_D // _LANES):
                    sl = pl.ds(j * _LANES, _LANES)
                    xbuf[p, bb, r, sl] = xbuf[p, bb, r, sl] + pbuf[p, r, sl]
                return c2
            return lax.fori_loop(0, _C, row_body, c3)
        lax.fori_loop(0, _BATCH, batch_body, 0)

    for c in range(_NBUF - 1):
        start_in(c, c)

    def step(si, carry):
        for k in range(_NBUF):
            c = si * _NBUF + k
            p = k
            r = (k + _NBUF - 1) % _NBUF

            @pl.when((c >= 1) & (c + _NBUF - 1 < _CHUNKS))
            def _():
                wait_out(r)

            @pl.when(c + _NBUF - 1 < _CHUNKS)
            def _():
                start_in(c + _NBUF - 1, r)

            wait_in(p)
            compute(p)
            start_out(c, p)
        return carry

    lax.fori_loop(0, _CHUNKS // _NBUF, step, 0)
    for p in range(_NBUF):
        wait_out(p)


def kernel(x, pos_weight):
    mesh = plsc.VectorSubcoreMesh(core_axis_name="c", subcore_axis_name="s")
    fn = functools.partial(
        pl.kernel,
        mesh=mesh,
        out_type=jax.ShapeDtypeStruct((_BATCH, _SEQ, _D), jnp.float32),
        scratch_types=[
            pltpu.VMEM((_NBUF, _BATCH, _C, _D), jnp.float32),
            pltpu.VMEM((_NBUF, _C, _D), jnp.float32),
        ] + [pltpu.SemaphoreType.DMA] * (2 * _NBUF),
    )(_sc_body)
    return fn(x, pos_weight)


# speedup vs baseline: 2.0967x; 2.0967x over previous
import functools

import jax
import jax.numpy as jnp
from jax import lax
from jax.experimental import pallas as pl
from jax.experimental.pallas import tpu as pltpu
from jax.experimental.pallas import tpu_sc as plsc

_BATCH = 4
_SEQ = 8192
_D = 1024
_LANES = 16
_NC = 2
_NS = 16
_NW = _NC * _NS

_SEQ_SC = 2048
_ROWS_PER_W = _SEQ_SC // _NW
_C = 8
_CHUNKS = _ROWS_PER_W // _C

_TC_BLOCK_S = 2048


def _sc_body(x_hbm, pos_hbm, out_hbm, xbuf, pbuf,
             in0, in1, out0, out1):
    wid = lax.axis_index("s") * _NC + lax.axis_index("c")
    s_base = wid * _ROWS_PER_W
    insem = (in0, in1)
    outsem = (out0, out1)

    def start_in(c, p):
        s0 = s_base + c * _C
        pltpu.async_copy(pos_hbm.at[pl.ds(s0, _C)], pbuf.at[p], insem[p])
        pltpu.async_copy(x_hbm.at[:, pl.ds(s0, _C)], xbuf.at[p], insem[p])

    def wait_in(p):
        pltpu.make_async_copy(pos_hbm.at[pl.ds(0, _C)], pbuf.at[p],
                              insem[p]).wait()
        pltpu.make_async_copy(x_hbm.at[:, pl.ds(0, _C)], xbuf.at[p],
                              insem[p]).wait()

    def start_out(c, p):
        s0 = s_base + c * _C
        pltpu.async_copy(xbuf.at[p], out_hbm.at[:, pl.ds(s0, _C)], outsem[p])

    def wait_out(p):
        pltpu.make_async_copy(x_hbm.at[:, pl.ds(0, _C)], xbuf.at[p],
                              outsem[p]).wait()

    def compute(p):
        for b in range(_BATCH):
            def row_body(r, c2, _b=b):
                for j in range(_D // _LANES):
                    sl = pl.ds(j * _LANES, _LANES)
                    xbuf[p, _b, r, sl] = xbuf[p, _b, r, sl] + pbuf[p, r, sl]
                return c2
            lax.fori_loop(0, _C, row_body, 0)

    start_in(0, 0)

    def step(si, carry):
        for p in range(2):
            c = si * 2 + p
            q = 1 - p

            @pl.when(c >= 1)
            def _():
                wait_out(q)

            @pl.when(c + 1 < _CHUNKS)
            def _():
                start_in(c + 1, q)

            wait_in(p)
            compute(p)
            start_out(c, p)
        return carry

    lax.fori_loop(0, _CHUNKS // 2, step, 0)
    wait_out(1)


def _sc_part(x_sc, pos_sc):
    mesh = plsc.VectorSubcoreMesh(core_axis_name="c", subcore_axis_name="s")
    fn = functools.partial(
        pl.kernel,
        mesh=mesh,
        out_type=jax.ShapeDtypeStruct((_BATCH, _SEQ_SC, _D), jnp.float32),
        scratch_types=[
            pltpu.VMEM((2, _BATCH, _C, _D), jnp.float32),
            pltpu.VMEM((2, _C, _D), jnp.float32),
            pltpu.SemaphoreType.DMA,
            pltpu.SemaphoreType.DMA,
            pltpu.SemaphoreType.DMA,
            pltpu.SemaphoreType.DMA,
        ],
    )(_sc_body)
    return fn(x_sc, pos_sc)


def _tc_add_kernel(x_ref, pos_ref, o_ref):
    o_ref[...] = x_ref[...] + pos_ref[...]


def _tc_part(x, pos_weight):
    seq_tc = _SEQ - _SEQ_SC
    off = _SEQ_SC // _TC_BLOCK_S
    grid = (seq_tc // _TC_BLOCK_S, _BATCH)
    return pl.pallas_call(
        _tc_add_kernel,
        grid=grid,
        in_specs=[
            pl.BlockSpec((1, _TC_BLOCK_S, _D), lambda s, b: (b, s + off, 0)),
            pl.BlockSpec((_TC_BLOCK_S, _D), lambda s, b: (s + off, 0)),
        ],
        out_specs=pl.BlockSpec((1, _TC_BLOCK_S, _D), lambda s, b: (b, s, 0)),
        out_shape=jax.ShapeDtypeStruct((_BATCH, seq_tc, _D), x.dtype),
        compiler_params=pltpu.CompilerParams(
            dimension_semantics=("arbitrary", "arbitrary"),
        ),
    )(x, pos_weight)


def kernel(x, pos_weight):
    out_sc = _sc_part(x, pos_weight)
    out_tc = _tc_part(x, pos_weight)
    return jnp.concatenate([out_sc, out_tc], axis=1)


# speedup vs baseline: 2.3169x; 1.1050x over previous
import functools

import jax
import jax.numpy as jnp
from jax import lax
from jax.experimental import pallas as pl
from jax.experimental.pallas import tpu as pltpu
from jax.experimental.pallas import tpu_sc as plsc

_BATCH = 4
_SEQ = 8192
_D = 1024
_LANES = 16
_NC = 2
_NS = 16
_NW = _NC * _NS
_ROWS_PER_W = _SEQ // _NW
_C = 8
_H = _C // 2
_CHUNKS = _ROWS_PER_W // _C


def _sc_body(x_hbm, pos_hbm, out_hbm, xbuf, pbuf,
             in0, in1, out0, out1):
    wid = lax.axis_index("s") * _NC + lax.axis_index("c")
    s_base = wid * _ROWS_PER_W
    insem = (in0, in1)
    outsem = (out0, out1)

    def start_in(c, p):
        s0 = s_base + c * _C
        pltpu.async_copy(pos_hbm.at[pl.ds(s0, _C)], pbuf.at[p], insem[p])
        pltpu.async_copy(x_hbm.at[:, pl.ds(s0, _C)], xbuf.at[p], insem[p])

    def wait_in(p):
        pltpu.make_async_copy(pos_hbm.at[pl.ds(0, _C)], pbuf.at[p],
                              insem[p]).wait()
        pltpu.make_async_copy(x_hbm.at[:, pl.ds(0, _C)], xbuf.at[p],
                              insem[p]).wait()

    def start_out_half(c, p, h):
        s0 = s_base + c * _C + h * _H
        pltpu.async_copy(xbuf.at[p, :, pl.ds(h * _H, _H)],
                         out_hbm.at[:, pl.ds(s0, _H)], outsem[p])

    def wait_out(p):
        pltpu.make_async_copy(x_hbm.at[:, pl.ds(0, _C)], xbuf.at[p],
                              outsem[p]).wait()

    def compute_half(p, h):
        for b in range(_BATCH):
            def row_body(r, c2, _b=b):
                rr = h * _H + r
                for j in range(_D // _LANES):
                    sl = pl.ds(j * _LANES, _LANES)
                    xbuf[p, _b, rr, sl] = xbuf[p, _b, rr, sl] + pbuf[p, rr, sl]
                return c2
            lax.fori_loop(0, _H, row_body, 0)

    start_in(0, 0)

    def step(si, carry):
        for p in range(2):
            c = si * 2 + p
            q = 1 - p

            @pl.when(c >= 1)
            def _():
                wait_out(q)

            @pl.when(c + 1 < _CHUNKS)
            def _():
                start_in(c + 1, q)

            wait_in(p)
            compute_half(p, 0)
            start_out_half(c, p, 0)
            compute_half(p, 1)
            start_out_half(c, p, 1)
        return carry

    lax.fori_loop(0, _CHUNKS // 2, step, 0)
    wait_out(1)


def kernel(x, pos_weight):
    mesh = plsc.VectorSubcoreMesh(core_axis_name="c", subcore_axis_name="s")
    fn = functools.partial(
        pl.kernel,
        mesh=mesh,
        out_type=jax.ShapeDtypeStruct((_BATCH, _SEQ, _D), jnp.float32),
        scratch_types=[
            pltpu.VMEM((2, _BATCH, _C, _D), jnp.float32),
            pltpu.VMEM((2, _C, _D), jnp.float32),
            pltpu.SemaphoreType.DMA,
            pltpu.SemaphoreType.DMA,
            pltpu.SemaphoreType.DMA,
            pltpu.SemaphoreType.DMA,
        ],
    )(_sc_body)
    return fn(x, pos_weight)


# speedup vs baseline: 2.8048x; 1.2106x over previous
import functools

import jax
import jax.numpy as jnp
from jax import lax
from jax.experimental import pallas as pl
from jax.experimental.pallas import tpu as pltpu
from jax.experimental.pallas import tpu_sc as plsc

_BATCH = 4
_SEQ = 8192
_D = 1024
_LANES = 16
_NC = 2
_NS = 16
_NW = _NC * _NS
_ROWS_PER_W = _SEQ // _NW
_C = 8
_CHUNKS = _ROWS_PER_W // _C
_NBUF = 3
_LOOP_CHUNKS = (_CHUNKS // _NBUF) * _NBUF


def _sc_body(x_hbm, pos_hbm, out_hbm, xbuf, pbuf, *sems):
    wid = lax.axis_index("s") * _NC + lax.axis_index("c")
    s_base = wid * _ROWS_PER_W
    insem = sems[:_NBUF]
    outsem = sems[_NBUF:]

    def start_in(c, p):
        s0 = s_base + c * _C
        pltpu.async_copy(pos_hbm.at[pl.ds(s0, _C)], pbuf.at[p], insem[p])
        pltpu.async_copy(x_hbm.at[:, pl.ds(s0, _C)], xbuf.at[p], insem[p])

    def wait_in(p):
        pltpu.make_async_copy(pos_hbm.at[pl.ds(0, _C)], pbuf.at[p],
                              insem[p]).wait()
        pltpu.make_async_copy(x_hbm.at[:, pl.ds(0, _C)], xbuf.at[p],
                              insem[p]).wait()

    def start_out(c, p):
        s0 = s_base + c * _C
        pltpu.async_copy(xbuf.at[p], out_hbm.at[:, pl.ds(s0, _C)], outsem[p])

    def wait_out(p):
        pltpu.make_async_copy(x_hbm.at[:, pl.ds(0, _C)], xbuf.at[p],
                              outsem[p]).wait()

    def compute(p):
        for b in range(_BATCH):
            def row_body(r, c2, _b=b):
                for j in range(_D // _LANES):
                    sl = pl.ds(j * _LANES, _LANES)
                    xbuf[p, _b, r, sl] = xbuf[p, _b, r, sl] + pbuf[p, r, sl]
                return c2
            lax.fori_loop(0, _C, row_body, 0)

    def body(c, p, q, first, last):
        if not first:
            @pl.when(c >= 2)
            def _():
                wait_out(q)
        if not last:
            @pl.when(c + 1 < _CHUNKS)
            def _():
                start_in(c + 1, q)
        wait_in(p)
        compute(p)
        start_out(c, p)

    start_in(0, 0)

    def step(si, carry):
        for k in range(_NBUF):
            c = si * _NBUF + k
            body(c, k, (k + 1) % _NBUF, first=False, last=False)
        return carry

    lax.fori_loop(0, _LOOP_CHUNKS // _NBUF, step, 0)
    for c in range(_LOOP_CHUNKS, _CHUNKS):
        body(c, c % _NBUF, (c + 1) % _NBUF, first=False, last=(c + 1 == _CHUNKS))
    wait_out(_LOOP_CHUNKS % _NBUF)
    wait_out((_LOOP_CHUNKS + 1) % _NBUF)


def kernel(x, pos_weight):
    mesh = plsc.VectorSubcoreMesh(core_axis_name="c", subcore_axis_name="s")
    fn = functools.partial(
        pl.kernel,
        mesh=mesh,
        out_type=jax.ShapeDtypeStruct((_BATCH, _SEQ, _D), jnp.float32),
        scratch_types=[
            pltpu.VMEM((_NBUF, _BATCH, _C, _D), jnp.float32),
            pltpu.VMEM((_NBUF, _C, _D), jnp.float32),
        ] + [pltpu.SemaphoreType.DMA] * (2 * _NBUF),
    )(_sc_body)
    return fn(x, pos_weight)


# speedup vs baseline: 2.8286x; 1.0085x over previous
import functools

import jax
import jax.numpy as jnp
from jax import lax
from jax.experimental import pallas as pl
from jax.experimental.pallas import tpu as pltpu
from jax.experimental.pallas import tpu_sc as plsc

_BATCH = 4
_SEQ = 8192
_D = 1024
_LANES = 16
_NC = 2
_NS = 16
_NW = _NC * _NS
_ROWS_PER_W = _SEQ // _NW
_C = 8
_CHUNKS = _ROWS_PER_W // _C
_NBUF = 3
_LOOP_CHUNKS = (_CHUNKS // _NBUF) * _NBUF


def _sc_body(x_hbm, pos_hbm, out_hbm, xbuf, pbuf, *sems):
    wid = lax.axis_index("s") * _NC + lax.axis_index("c")
    s_base = wid * _ROWS_PER_W
    insem = sems[:_NBUF]
    outsem = sems[_NBUF:]

    def start_in(c, p):
        s0 = s_base + c * _C
        pltpu.async_copy(pos_hbm.at[pl.ds(s0, _C)], pbuf.at[p], insem[p])
        pltpu.async_copy(x_hbm.at[:, pl.ds(s0, _C)], xbuf.at[p], insem[p])

    def wait_in(p):
        pltpu.make_async_copy(pos_hbm.at[pl.ds(0, _C)], pbuf.at[p],
                              insem[p]).wait()
        pltpu.make_async_copy(x_hbm.at[:, pl.ds(0, _C)], xbuf.at[p],
                              insem[p]).wait()

    def wait_out(p):
        pltpu.make_async_copy(x_hbm.at[:, pl.ds(0, _C)], xbuf.at[p],
                              outsem[p]).wait()

    def compute_and_store(c, p):
        s0 = s_base + c * _C
        for b in range(_BATCH):
            def row_body(r, c2, _b=b):
                for j in range(_D // _LANES):
                    sl = pl.ds(j * _LANES, _LANES)
                    xbuf[p, _b, r, sl] = xbuf[p, _b, r, sl] + pbuf[p, r, sl]
                return c2
            lax.fori_loop(0, _C, row_body, 0)
            pltpu.async_copy(xbuf.at[p, b], out_hbm.at[b, pl.ds(s0, _C)],
                             outsem[p])

    def body(c, p, q, first, last):
        if not first:
            @pl.when(c >= 2)
            def _():
                wait_out(q)
        if not last:
            @pl.when(c + 1 < _CHUNKS)
            def _():
                start_in(c + 1, q)
        wait_in(p)
        compute_and_store(c, p)

    start_in(0, 0)

    def step(si, carry):
        for k in range(_NBUF):
            c = si * _NBUF + k
            body(c, k, (k + 1) % _NBUF, first=False, last=False)
        return carry

    lax.fori_loop(0, _LOOP_CHUNKS // _NBUF, step, 0)
    for c in range(_LOOP_CHUNKS, _CHUNKS):
        body(c, c % _NBUF, (c + 1) % _NBUF, first=False, last=(c + 1 == _CHUNKS))
    wait_out(_LOOP_CHUNKS % _NBUF)
    wait_out((_LOOP_CHUNKS + 1) % _NBUF)


def kernel(x, pos_weight):
    mesh = plsc.VectorSubcoreMesh(core_axis_name="c", subcore_axis_name="s")
    fn = functools.partial(
        pl.kernel,
        mesh=mesh,
        out_type=jax.ShapeDtypeStruct((_BATCH, _SEQ, _D), jnp.float32),
        scratch_types=[
            pltpu.VMEM((_NBUF, _BATCH, _C, _D), jnp.float32),
            pltpu.VMEM((_NBUF, _C, _D), jnp.float32),
        ] + [pltpu.SemaphoreType.DMA] * (2 * _NBUF),
    )(_sc_body)
    return fn(x, pos_weight)


# speedup vs baseline: 2.9269x; 1.0348x over previous
import functools

import jax
import jax.numpy as jnp
from jax import lax
from jax.experimental import pallas as pl
from jax.experimental.pallas import tpu as pltpu
from jax.experimental.pallas import tpu_sc as plsc

_BATCH = 4
_SEQ = 8192
_D = 1024
_LANES = 16
_NC = 2
_NS = 16
_NW = _NC * _NS
_ROWS_PER_W = _SEQ // _NW
_C = 8
_CHUNKS = _ROWS_PER_W // _C
_NBUF = 3
_LOOP_CHUNKS = (_CHUNKS // _NBUF) * _NBUF


def _sc_body(x_hbm, pos_hbm, out_hbm, xbuf, pbuf, *sems):
    wid = lax.axis_index("s") * _NC + lax.axis_index("c")
    s_base = wid * _ROWS_PER_W
    insem = sems[:_NBUF]
    outsem = sems[_NBUF:]

    def start_in(c, p):
        s0 = s_base + c * _C
        pltpu.async_copy(pos_hbm.at[pl.ds(s0, _C)], pbuf.at[p], insem[p])
        pltpu.async_copy(x_hbm.at[:, pl.ds(s0, _C)], xbuf.at[p], insem[p])

    def wait_in(p):
        pltpu.make_async_copy(pos_hbm.at[pl.ds(0, _C)], pbuf.at[p],
                              insem[p]).wait()
        pltpu.make_async_copy(x_hbm.at[:, pl.ds(0, _C)], xbuf.at[p],
                              insem[p]).wait()

    def wait_out(p):
        pltpu.make_async_copy(x_hbm.at[:, pl.ds(0, _C)], xbuf.at[p],
                              outsem[p]).wait()

    def compute_and_store(c, p):
        s0 = s_base + c * _C
        for b in range(_BATCH):
            def row_body(r, c2, _b=b):
                for j in range(_D // _LANES):
                    sl = pl.ds(j * _LANES, _LANES)
                    plsc.addupdate(xbuf.at[p, _b, r, sl], pbuf[p, r, sl])
                return c2
            lax.fori_loop(0, _C, row_body, 0)
            pltpu.async_copy(xbuf.at[p, b], out_hbm.at[b, pl.ds(s0, _C)],
                             outsem[p])

    def body(c, p, q, first, last):
        if not first:
            @pl.when(c >= 2)
            def _():
                wait_out(q)
        if not last:
            @pl.when(c + 1 < _CHUNKS)
            def _():
                start_in(c + 1, q)
        wait_in(p)
        compute_and_store(c, p)

    start_in(0, 0)

    def step(si, carry):
        for k in range(_NBUF):
            c = si * _NBUF + k
            body(c, k, (k + 1) % _NBUF, first=False, last=False)
        return carry

    lax.fori_loop(0, _LOOP_CHUNKS // _NBUF, step, 0)
    for c in range(_LOOP_CHUNKS, _CHUNKS):
        body(c, c % _NBUF, (c + 1) % _NBUF, first=False, last=(c + 1 == _CHUNKS))
    wait_out(_LOOP_CHUNKS % _NBUF)
    wait_out((_LOOP_CHUNKS + 1) % _NBUF)


def kernel(x, pos_weight):
    mesh = plsc.VectorSubcoreMesh(core_axis_name="c", subcore_axis_name="s")
    fn = functools.partial(
        pl.kernel,
        mesh=mesh,
        out_type=jax.ShapeDtypeStruct((_BATCH, _SEQ, _D), jnp.float32),
        scratch_types=[
            pltpu.VMEM((_NBUF, _BATCH, _C, _D), jnp.float32),
            pltpu.VMEM((_NBUF, _C, _D), jnp.float32),
        ] + [pltpu.SemaphoreType.DMA] * (2 * _NBUF),
    )(_sc_body)
    return fn(x, pos_weight)
